# baseline (speedup 1.0000x reference)
import functools

import jax
import jax.numpy as jnp
from jax import lax
from jax.experimental import pallas as pl
from jax.experimental.pallas import tpu as pltpu
from jax.experimental.pallas import tpu_sc as plsc

B = 16
N = 4096
S = 512
K = 32
CIN = 6
RADIUS_SQ = 0.04
M = K * B * S

_HIGHEST = jax.lax.Precision.HIGHEST


def _rbf16(v):
    u = jax.lax.bitcast_convert_type(v, jnp.int32)
    r = (u + 0x7FFF + ((u >> 16) & 1)) & jnp.int32(-65536)
    return jax.lax.bitcast_convert_type(r, jnp.float32)


def _dot(w, x):
    return jax.lax.dot_general(
        _rbf16(w), _rbf16(x), (((1,), (0,)), ((), ())),
        preferred_element_type=jnp.float32)


_FB = 2
_FR = 8
_FC = N // _FR


def _fps_body(xyz_ref, newxyz_ref):
    x = xyz_ref[:, 0]
    y = xyz_ref[:, 1]
    z = xyz_ref[:, 2]
    shp = (_FB, _FR, _FC)
    ii = (jax.lax.broadcasted_iota(jnp.int32, shp, 1) * _FC
          + jax.lax.broadcasted_iota(jnp.int32, shp, 2))
    iota_s = jax.lax.broadcasted_iota(jnp.int32, (_FB, S), 1)

    def _rmin(a):
        return jnp.min(jnp.min(a, axis=2, keepdims=True), axis=1,
                       keepdims=True)

    def _rmax(a):
        return jnp.max(jnp.max(a, axis=2, keepdims=True), axis=1,
                       keepdims=True)

    def _rsum(a):
        return jnp.sum(jnp.sum(a, axis=2, keepdims=True), axis=1,
                       keepdims=True)

    def body(i, carry):
        dist, idx, cxs, cys, czs = carry
        onehot = ii == idx
        cx = _rsum(jnp.where(onehot, x, 0.0))
        cy = _rsum(jnp.where(onehot, y, 0.0))
        cz = _rsum(jnp.where(onehot, z, 0.0))
        colmask = iota_s == i
        cxs = jnp.where(colmask, cx[:, :, 0], cxs)
        cys = jnp.where(colmask, cy[:, :, 0], cys)
        czs = jnp.where(colmask, cz[:, :, 0], czs)
        dx = x - cx
        dy = y - cy
        dz = z - cz
        d = (dx * dx + dy * dy) + dz * dz
        dist = jnp.minimum(dist, d)
        m = _rmax(dist)
        idxn = _rmin(jnp.where(dist == m, ii, N))
        return dist, idxn, cxs, cys, czs

    dist0 = jnp.full(shp, 1e10, dtype=jnp.float32)
    idx0 = jnp.zeros((_FB, 1, 1), dtype=jnp.int32)
    cs0 = jnp.zeros((_FB, S), dtype=jnp.float32)
    _, _, cxs, cys, czs = lax.fori_loop(
        0, S, body, (dist0, idx0, cs0, cs0, cs0))
    newxyz_ref[:, 0, :] = cxs
    newxyz_ref[:, 1, :] = cys
    newxyz_ref[:, 2, :] = czs


def _fps_call(xyz):
    return pl.pallas_call(
        _fps_body,
        grid=(B // _FB,),
        in_specs=[pl.BlockSpec((_FB, 3, _FR, _FC), lambda i: (i, 0, 0, 0))],
        out_specs=pl.BlockSpec((_FB, 3, S), lambda i: (i, 0, 0)),
        out_shape=jax.ShapeDtypeStruct((B, 3, S), jnp.float32),
    )(xyz.reshape(B, 3, _FR, _FC))


@functools.cache
def _group_call():
    mesh = plsc.VectorSubcoreMesh(core_axis_name="c", subcore_axis_name="s")
    return pl.kernel(
        _group_body,
        out_type=jax.ShapeDtypeStruct((CIN * K * B * S,), jnp.float32),
        mesh=mesh,
        compiler_params=pltpu.CompilerParams(needs_layout_passes=False),
        scratch_types=[
            pltpu.VMEM((CIN * N,), jnp.float32),
            pltpu.VMEM((3 * N,), jnp.float32),
            pltpu.VMEM((N,), jnp.float32),
            pltpu.VMEM((3 * S,), jnp.float32),
            pltpu.VMEM((48,), jnp.int32),
            pltpu.VMEM((CIN * K * (S // 2),), jnp.float32),
        ],
    )


_SHALF = S // 2


def _rbf16_sc(v):
    u = plsc.bitcast(v, jnp.int32)
    r = (u + 0x7FFF + ((u >> 16) & 1)) & jnp.int32(-65536)
    return plsc.bitcast(r, jnp.float32)


def _group_body(xyz_hbm, pts_hbm, nxyz_hbm, out_hbm,
                pts_v, rpts_v, pn_v, cent_v, idx_v, stage_v):
    b = lax.axis_index("s")
    half = lax.axis_index("c")
    s0 = half * _SHALF

    for c in range(3):
        pltpu.sync_copy(
            xyz_hbm.at[pl.ds(pl.multiple_of(b * (3 * N) + c * N, 8), N)],
            pts_v.at[pl.ds(c * N, N)])
        pltpu.sync_copy(
            pts_hbm.at[pl.ds(pl.multiple_of(b * (3 * N) + c * N, 8), N)],
            pts_v.at[pl.ds((3 + c) * N, N)])
        pltpu.sync_copy(
            nxyz_hbm.at[pl.ds(pl.multiple_of(b * (3 * S) + c * S, 8), S)],
            cent_v.at[pl.ds(c * S, S)])

    lane = jax.lax.broadcasted_iota(jnp.int32, (16,), 0)

    @plsc.parallel_loop(0, N // 16, unroll=8)
    def pn_chunk(j):
        base = pl.multiple_of(j * 16, 16)
        xv = pts_v[pl.ds(base, 16)]
        yv = pts_v[pl.ds(base + N, 16)]
        zv = pts_v[pl.ds(base + 2 * N, 16)]
        pn_v[pl.ds(base, 16)] = (xv * xv + yv * yv) + zv * zv
        rpts_v[pl.ds(base, 16)] = _rbf16_sc(xv)
        rpts_v[pl.ds(base + N, 16)] = _rbf16_sc(yv)
        rpts_v[pl.ds(base + 2 * N, 16)] = _rbf16_sc(zv)

    pad = jnp.full((16,), N - 1, dtype=jnp.int32)
    zeros16 = jnp.full((16,), 0, dtype=jnp.int32)

    def centroid_body(sl, _):
        s = s0 + sl
        cxv = plsc.load_gather(cent_v, [zeros16 + s])
        cyv = plsc.load_gather(cent_v, [zeros16 + (s + S)])
        czv = plsc.load_gather(cent_v, [zeros16 + (s + 2 * S)])
        cx, cy, cz = cxv[0], cyv[0], czv[0]
        cxb = _rbf16_sc(cxv)[0]
        cyb = _rbf16_sc(cyv)[0]
        czb = _rbf16_sc(czv)[0]
        cn = (cx * cx + cy * cy) + cz * cz
        idx_v[pl.ds(0, 16)] = pad
        idx_v[pl.ds(16, 16)] = pad
        idx_v[pl.ds(32, 16)] = pad

        @plsc.parallel_loop(0, N // 16, unroll=8, carry=jnp.int32(0))
        def chunk(j, cnt):
            base = pl.multiple_of(j * 16, 16)
            xv = rpts_v[pl.ds(base, 16)]
            yv = rpts_v[pl.ds(base + N, 16)]
            zv = rpts_v[pl.ds(base + 2 * N, 16)]
            pnv = pn_v[pl.ds(base, 16)]
            t = (cxb * xv + cyb * yv) + czb * zv
            d = (-2.0 * t + cn) + pnv
            mask = d <= RADIUS_SQ
            cs = plsc.cumsum(mask.astype(jnp.int32))
            pos = jnp.minimum(cnt + cs - 1, 47)
            plsc.store_scatter(idx_v, [pos], lane + base, mask=mask)
            return cnt + cs[15]

        for h in range(2):
            giv = idx_v[pl.ds(h * 16, 16)]
            posb = (lane + h * 16) * _SHALF + sl
            for c in range(3):
                g = plsc.load_gather(pts_v, [giv + c * N])
                cc = (cx, cy, cz)[c]
                plsc.store_scatter(stage_v, [posb + c * (K * _SHALF)], g - cc)
            for c in range(3, 6):
                g = plsc.load_gather(pts_v, [giv + c * N])
                plsc.store_scatter(stage_v, [posb + c * (K * _SHALF)], g)
        return 0

    lax.fori_loop(0, _SHALF, centroid_body, 0)

    for c in range(CIN):
        for k in range(K):
            dst = (c * K + k) * (B * S) + b * S + s0
            pltpu.sync_copy(
                stage_v.at[pl.ds((c * K + k) * _SHALF, _SHALF)],
                out_hbm.at[pl.ds(pl.multiple_of(dst, 8), _SHALF)])


def _affine(y, sc_ref, sh_ref):
    return jnp.maximum(y * sc_ref[...] + sh_ref[...], 0.0)


def _stats(y):
    return (jnp.sum(y, axis=1, keepdims=True),
            jnp.sum(y * y, axis=1, keepdims=True))


def _acc_out(s_ref, q_ref, ps, pq):
    @pl.when(pl.program_id(0) == 0)
    def _():
        s_ref[...] = jnp.zeros_like(s_ref)
        q_ref[...] = jnp.zeros_like(q_ref)
    s_ref[...] += ps
    q_ref[...] += pq


def _p1_body(x_ref, w1_ref, b1_ref, s_ref, q_ref):
    y1 = _dot(w1_ref[...], x_ref[...]) + b1_ref[...]
    _acc_out(s_ref, q_ref, *_stats(y1))


def _p2_body(x_ref, w1_ref, b1_ref, sc1_ref, sh1_ref, w2_ref, b2_ref,
             s_ref, q_ref):
    y1 = _dot(w1_ref[...], x_ref[...]) + b1_ref[...]
    z1 = _affine(y1, sc1_ref, sh1_ref)
    y2 = _dot(w2_ref[...], z1) + b2_ref[...]
    _acc_out(s_ref, q_ref, *_stats(y2))


def _p3_body(x_ref, w1_ref, b1_ref, sc1_ref, sh1_ref, w2_ref, b2_ref,
             sc2_ref, sh2_ref, w3_ref, b3_ref, s_ref, q_ref):
    y1 = _dot(w1_ref[...], x_ref[...]) + b1_ref[...]
    z1 = _affine(y1, sc1_ref, sh1_ref)
    y2 = _dot(w2_ref[...], z1) + b2_ref[...]
    z2 = _affine(y2, sc2_ref, sh2_ref)
    y3 = _dot(w3_ref[...], z2) + b3_ref[...]
    _acc_out(s_ref, q_ref, *_stats(y3))


def _p4_body(x_ref, w1_ref, b1_ref, sc1_ref, sh1_ref, w2_ref, b2_ref,
             sc2_ref, sh2_ref, w3_ref, b3_ref, sc3_ref, sh3_ref, o_ref):
    acc = None
    for k in range(K):
        xk = x_ref[:, k, :]
        y1 = _dot(w1_ref[...], xk) + b1_ref[...]
        z1 = _affine(y1, sc1_ref, sh1_ref)
        y2 = _dot(w2_ref[...], z1) + b2_ref[...]
        z2 = _affine(y2, sc2_ref, sh2_ref)
        y3 = _dot(w3_ref[...], z2) + b3_ref[...]
        z3 = _affine(y3, sc3_ref, sh3_ref)
        acc = z3 if acc is None else jnp.maximum(acc, z3)
    o_ref[...] = acc


_MB = 2048
_LB = 1024


def _col_spec(rows, mb):
    return pl.BlockSpec((rows, mb), lambda i: (0, i))


def _full_spec(shape):
    return pl.BlockSpec(shape, lambda i: tuple(0 for _ in shape))


def _stat_specs(c):
    return [pl.BlockSpec((c, 1), lambda i: (0, 0))] * 2


def _stats_call(body, x, args, c, extra_specs):
    grid = (x.shape[1] // _MB,)
    out = pl.pallas_call(
        body,
        grid=grid,
        in_specs=[_col_spec(CIN, _MB)] + extra_specs,
        out_specs=_stat_specs(c),
        out_shape=[jax.ShapeDtypeStruct((c, 1), jnp.float32)] * 2,
    )(x, *args)
    return out


def _finalize(s, q, g, bt):
    mean = s[:, 0] / M
    var = q[:, 0] / M - mean * mean
    rstd = jax.lax.rsqrt(var + 1e-5)
    scale = g * rstd
    shift = bt - mean * scale
    return scale[:, None], shift[:, None]


def _mlp_call(x_flat, w1, b1, g1, bt1, w2, b2, g2, bt2, w3, b3, g3, bt3):
    b1c, b2c, b3c = b1[:, None], b2[:, None], b3[:, None]
    s1, q1 = _stats_call(_p1_body, x_flat,
                         (w1, b1c), 64,
                         [_full_spec((64, CIN)), _full_spec((64, 1))])
    sc1, sh1 = _finalize(s1, q1, g1, bt1)
    s2, q2 = _stats_call(_p2_body, x_flat,
                         (w1, b1c, sc1, sh1, w2, b2c), 64,
                         [_full_spec((64, CIN)), _full_spec((64, 1)),
                          _full_spec((64, 1)), _full_spec((64, 1)),
                          _full_spec((64, 64)), _full_spec((64, 1))])
    sc2, sh2 = _finalize(s2, q2, g2, bt2)
    s3, q3 = _stats_call(_p3_body, x_flat,
                         (w1, b1c, sc1, sh1, w2, b2c, sc2, sh2, w3, b3c), 128,
                         [_full_spec((64, CIN)), _full_spec((64, 1)),
                          _full_spec((64, 1)), _full_spec((64, 1)),
                          _full_spec((64, 64)), _full_spec((64, 1)),
                          _full_spec((64, 1)), _full_spec((64, 1)),
                          _full_spec((128, 64)), _full_spec((128, 1))])
    sc3, sh3 = _finalize(s3, q3, g3, bt3)

    x3d = x_flat.reshape(CIN, K, B * S)
    pooled = pl.pallas_call(
        _p4_body,
        grid=(B * S // _LB,),
        in_specs=[pl.BlockSpec((CIN, K, _LB), lambda i: (0, 0, i)),
                  _full_spec((64, CIN)), _full_spec((64, 1)),
                  _full_spec((64, 1)), _full_spec((64, 1)),
                  _full_spec((64, 64)), _full_spec((64, 1)),
                  _full_spec((64, 1)), _full_spec((64, 1)),
                  _full_spec((128, 64)), _full_spec((128, 1)),
                  _full_spec((128, 1)), _full_spec((128, 1))],
        out_specs=_col_spec(128, _LB),
        out_shape=jax.ShapeDtypeStruct((128, B * S), jnp.float32),
    )(x3d, w1, b1c, sc1, sh1, w2, b2c, sc2, sh2, w3, b3c, sc3, sh3)
    return pooled


def kernel(xyz, points, w1, b1, g1, bt1, w2, b2, g2, bt2, w3, b3, g3, bt3):
    new_xyz = _fps_call(xyz)
    grouped = _group_call()(xyz.reshape(-1), points.reshape(-1),
                            new_xyz.reshape(-1))
    x_flat = grouped.reshape(CIN, M)
    pooled = _mlp_call(x_flat, w1, b1, g1, bt1, w2, b2, g2, bt2,
                       w3, b3, g3, bt3)
    new_points_out = pooled.reshape(128, B, S).transpose(1, 0, 2)
    return new_xyz, new_points_out


# speedup vs baseline: 2.1658x; 2.1658x over previous
import functools

import jax
import jax.numpy as jnp
from jax import lax
from jax.experimental import pallas as pl
from jax.experimental.pallas import tpu as pltpu
from jax.experimental.pallas import tpu_sc as plsc

B = 16
N = 4096
S = 512
K = 32
CIN = 6
RADIUS_SQ = 0.04
M = K * B * S

_HIGHEST = jax.lax.Precision.HIGHEST


def _rbf16(v):
    u = jax.lax.bitcast_convert_type(v, jnp.int32)
    r = (u + 0x7FFF + ((u >> 16) & 1)) & jnp.int32(-65536)
    return jax.lax.bitcast_convert_type(r, jnp.float32)


def _dot(w, x):
    return jax.lax.dot_general(
        _rbf16(w), _rbf16(x), (((1,), (0,)), ((), ())),
        preferred_element_type=jnp.float32)


_FB = 16
_FR = 8
_FC = N // _FR


def _fps_body(xyz_ref, newxyz_ref):
    x = xyz_ref[:, 0]
    y = xyz_ref[:, 1]
    z = xyz_ref[:, 2]
    shp = (_FB, _FR, _FC)
    ii = (jax.lax.broadcasted_iota(jnp.int32, shp, 1) * _FC
          + jax.lax.broadcasted_iota(jnp.int32, shp, 2))
    iota_s = jax.lax.broadcasted_iota(jnp.int32, (_FB, S), 1)

    def _rmin(a):
        return jnp.min(jnp.min(a, axis=2, keepdims=True), axis=1,
                       keepdims=True)

    def _rmax(a):
        return jnp.max(jnp.max(a, axis=2, keepdims=True), axis=1,
                       keepdims=True)

    def _rsum(a):
        return jnp.sum(jnp.sum(a, axis=2, keepdims=True), axis=1,
                       keepdims=True)

    def body(i, carry):
        dist, idx, cxs, cys, czs = carry
        onehot = ii == idx
        cx = _rsum(jnp.where(onehot, x, 0.0))
        cy = _rsum(jnp.where(onehot, y, 0.0))
        cz = _rsum(jnp.where(onehot, z, 0.0))
        colmask = iota_s == i
        cxs = jnp.where(colmask, cx[:, :, 0], cxs)
        cys = jnp.where(colmask, cy[:, :, 0], cys)
        czs = jnp.where(colmask, cz[:, :, 0], czs)
        dx = x - cx
        dy = y - cy
        dz = z - cz
        d = (dx * dx + dy * dy) + dz * dz
        dist = jnp.minimum(dist, d)
        m = _rmax(dist)
        idxn = _rmin(jnp.where(dist == m, ii, N))
        return dist, idxn, cxs, cys, czs

    dist0 = jnp.full(shp, 1e10, dtype=jnp.float32)
    idx0 = jnp.zeros((_FB, 1, 1), dtype=jnp.int32)
    cs0 = jnp.zeros((_FB, S), dtype=jnp.float32)
    _, _, cxs, cys, czs = lax.fori_loop(
        0, S, body, (dist0, idx0, cs0, cs0, cs0))
    newxyz_ref[:, 0, :] = cxs
    newxyz_ref[:, 1, :] = cys
    newxyz_ref[:, 2, :] = czs


def _fps_call(xyz):
    return pl.pallas_call(
        _fps_body,
        grid=(B // _FB,),
        in_specs=[pl.BlockSpec((_FB, 3, _FR, _FC), lambda i: (i, 0, 0, 0))],
        out_specs=pl.BlockSpec((_FB, 3, S), lambda i: (i, 0, 0)),
        out_shape=jax.ShapeDtypeStruct((B, 3, S), jnp.float32),
    )(xyz.reshape(B, 3, _FR, _FC))


@functools.cache
def _group_call():
    mesh = plsc.VectorSubcoreMesh(core_axis_name="c", subcore_axis_name="s")
    return pl.kernel(
        _group_body,
        out_type=jax.ShapeDtypeStruct((CIN * K * B * S,), jnp.float32),
        mesh=mesh,
        compiler_params=pltpu.CompilerParams(needs_layout_passes=False),
        scratch_types=[
            pltpu.VMEM((CIN * N,), jnp.float32),
            pltpu.VMEM((3 * N,), jnp.float32),
            pltpu.VMEM((N,), jnp.float32),
            pltpu.VMEM((3 * S,), jnp.float32),
            pltpu.VMEM((48,), jnp.int32),
            pltpu.VMEM((CIN * K * (S // 2),), jnp.float32),
        ],
    )


_SHALF = S // 2


def _rbf16_sc(v):
    u = plsc.bitcast(v, jnp.int32)
    r = (u + 0x7FFF + ((u >> 16) & 1)) & jnp.int32(-65536)
    return plsc.bitcast(r, jnp.float32)


def _group_body(xyz_hbm, pts_hbm, nxyz_hbm, out_hbm,
                pts_v, rpts_v, pn_v, cent_v, idx_v, stage_v):
    b = lax.axis_index("s")
    half = lax.axis_index("c")
    s0 = half * _SHALF

    for c in range(3):
        pltpu.sync_copy(
            xyz_hbm.at[pl.ds(pl.multiple_of(b * (3 * N) + c * N, 8), N)],
            pts_v.at[pl.ds(c * N, N)])
        pltpu.sync_copy(
            pts_hbm.at[pl.ds(pl.multiple_of(b * (3 * N) + c * N, 8), N)],
            pts_v.at[pl.ds((3 + c) * N, N)])
        pltpu.sync_copy(
            nxyz_hbm.at[pl.ds(pl.multiple_of(b * (3 * S) + c * S, 8), S)],
            cent_v.at[pl.ds(c * S, S)])

    lane = jax.lax.broadcasted_iota(jnp.int32, (16,), 0)

    @plsc.parallel_loop(0, N // 16, unroll=8)
    def pn_chunk(j):
        base = pl.multiple_of(j * 16, 16)
        xv = pts_v[pl.ds(base, 16)]
        yv = pts_v[pl.ds(base + N, 16)]
        zv = pts_v[pl.ds(base + 2 * N, 16)]
        pn_v[pl.ds(base, 16)] = (xv * xv + yv * yv) + zv * zv
        rpts_v[pl.ds(base, 16)] = _rbf16_sc(xv)
        rpts_v[pl.ds(base + N, 16)] = _rbf16_sc(yv)
        rpts_v[pl.ds(base + 2 * N, 16)] = _rbf16_sc(zv)

    pad = jnp.full((16,), N - 1, dtype=jnp.int32)
    zeros16 = jnp.full((16,), 0, dtype=jnp.int32)

    def centroid_body(sl, _):
        s = s0 + sl
        cxv = plsc.load_gather(cent_v, [zeros16 + s])
        cyv = plsc.load_gather(cent_v, [zeros16 + (s + S)])
        czv = plsc.load_gather(cent_v, [zeros16 + (s + 2 * S)])
        cx, cy, cz = cxv[0], cyv[0], czv[0]
        cxb = _rbf16_sc(cxv)[0]
        cyb = _rbf16_sc(cyv)[0]
        czb = _rbf16_sc(czv)[0]
        cn = (cx * cx + cy * cy) + cz * cz
        idx_v[pl.ds(0, 16)] = pad
        idx_v[pl.ds(16, 16)] = pad
        idx_v[pl.ds(32, 16)] = pad

        @plsc.parallel_loop(0, N // 16, unroll=8, carry=jnp.int32(0))
        def chunk(j, cnt):
            base = pl.multiple_of(j * 16, 16)
            xv = rpts_v[pl.ds(base, 16)]
            yv = rpts_v[pl.ds(base + N, 16)]
            zv = rpts_v[pl.ds(base + 2 * N, 16)]
            pnv = pn_v[pl.ds(base, 16)]
            t = (cxb * xv + cyb * yv) + czb * zv
            d = (-2.0 * t + cn) + pnv
            mask = d <= RADIUS_SQ
            cs = plsc.cumsum(mask.astype(jnp.int32))
            pos = jnp.minimum(cnt + cs - 1, 47)
            plsc.store_scatter(idx_v, [pos], lane + base, mask=mask)
            return cnt + cs[15]

        for h in range(2):
            giv = idx_v[pl.ds(h * 16, 16)]
            posb = (lane + h * 16) * _SHALF + sl
            for c in range(3):
                g = plsc.load_gather(pts_v, [giv + c * N])
                cc = (cx, cy, cz)[c]
                plsc.store_scatter(stage_v, [posb + c * (K * _SHALF)], g - cc)
            for c in range(3, 6):
                g = plsc.load_gather(pts_v, [giv + c * N])
                plsc.store_scatter(stage_v, [posb + c * (K * _SHALF)], g)
        return 0

    lax.fori_loop(0, _SHALF, centroid_body, 0)

    for c in range(CIN):
        for k in range(K):
            dst = (c * K + k) * (B * S) + b * S + s0
            pltpu.sync_copy(
                stage_v.at[pl.ds((c * K + k) * _SHALF, _SHALF)],
                out_hbm.at[pl.ds(pl.multiple_of(dst, 8), _SHALF)])


def _affine(y, sc_ref, sh_ref):
    return jnp.maximum(y * sc_ref[...] + sh_ref[...], 0.0)


def _stats(y):
    return (jnp.sum(y, axis=1, keepdims=True),
            jnp.sum(y * y, axis=1, keepdims=True))


def _acc_out(s_ref, q_ref, ps, pq):
    @pl.when(pl.program_id(0) == 0)
    def _():
        s_ref[...] = jnp.zeros_like(s_ref)
        q_ref[...] = jnp.zeros_like(q_ref)
    s_ref[...] += ps
    q_ref[...] += pq


def _p1_body(x_ref, w1_ref, b1_ref, s_ref, q_ref):
    y1 = _dot(w1_ref[...], x_ref[...]) + b1_ref[...]
    _acc_out(s_ref, q_ref, *_stats(y1))


def _p2_body(x_ref, w1_ref, b1_ref, sc1_ref, sh1_ref, w2_ref, b2_ref,
             s_ref, q_ref):
    y1 = _dot(w1_ref[...], x_ref[...]) + b1_ref[...]
    z1 = _affine(y1, sc1_ref, sh1_ref)
    y2 = _dot(w2_ref[...], z1) + b2_ref[...]
    _acc_out(s_ref, q_ref, *_stats(y2))


def _p3_body(x_ref, w1_ref, b1_ref, sc1_ref, sh1_ref, w2_ref, b2_ref,
             sc2_ref, sh2_ref, w3_ref, b3_ref, s_ref, q_ref):
    y1 = _dot(w1_ref[...], x_ref[...]) + b1_ref[...]
    z1 = _affine(y1, sc1_ref, sh1_ref)
    y2 = _dot(w2_ref[...], z1) + b2_ref[...]
    z2 = _affine(y2, sc2_ref, sh2_ref)
    y3 = _dot(w3_ref[...], z2) + b3_ref[...]
    _acc_out(s_ref, q_ref, *_stats(y3))


def _p4_body(x_ref, w1_ref, b1_ref, sc1_ref, sh1_ref, w2_ref, b2_ref,
             sc2_ref, sh2_ref, w3_ref, b3_ref, sc3_ref, sh3_ref, o_ref):
    acc = None
    for k in range(K):
        xk = x_ref[:, k, :]
        y1 = _dot(w1_ref[...], xk) + b1_ref[...]
        z1 = _affine(y1, sc1_ref, sh1_ref)
        y2 = _dot(w2_ref[...], z1) + b2_ref[...]
        z2 = _affine(y2, sc2_ref, sh2_ref)
        y3 = _dot(w3_ref[...], z2) + b3_ref[...]
        z3 = _affine(y3, sc3_ref, sh3_ref)
        acc = z3 if acc is None else jnp.maximum(acc, z3)
    o_ref[...] = acc


_MB = 2048
_LB = 1024


def _col_spec(rows, mb):
    return pl.BlockSpec((rows, mb), lambda i: (0, i))


def _full_spec(shape):
    return pl.BlockSpec(shape, lambda i: tuple(0 for _ in shape))


def _stat_specs(c):
    return [pl.BlockSpec((c, 1), lambda i: (0, 0))] * 2


def _stats_call(body, x, args, c, extra_specs):
    grid = (x.shape[1] // _MB,)
    out = pl.pallas_call(
        body,
        grid=grid,
        in_specs=[_col_spec(CIN, _MB)] + extra_specs,
        out_specs=_stat_specs(c),
        out_shape=[jax.ShapeDtypeStruct((c, 1), jnp.float32)] * 2,
    )(x, *args)
    return out


def _finalize(s, q, g, bt):
    mean = s[:, 0] / M
    var = q[:, 0] / M - mean * mean
    rstd = jax.lax.rsqrt(var + 1e-5)
    scale = g * rstd
    shift = bt - mean * scale
    return scale[:, None], shift[:, None]


def _mlp_call(x_flat, w1, b1, g1, bt1, w2, b2, g2, bt2, w3, b3, g3, bt3):
    b1c, b2c, b3c = b1[:, None], b2[:, None], b3[:, None]
    s1, q1 = _stats_call(_p1_body, x_flat,
                         (w1, b1c), 64,
                         [_full_spec((64, CIN)), _full_spec((64, 1))])
    sc1, sh1 = _finalize(s1, q1, g1, bt1)
    s2, q2 = _stats_call(_p2_body, x_flat,
                         (w1, b1c, sc1, sh1, w2, b2c), 64,
                         [_full_spec((64, CIN)), _full_spec((64, 1)),
                          _full_spec((64, 1)), _full_spec((64, 1)),
                          _full_spec((64, 64)), _full_spec((64, 1))])
    sc2, sh2 = _finalize(s2, q2, g2, bt2)
    s3, q3 = _stats_call(_p3_body, x_flat,
                         (w1, b1c, sc1, sh1, w2, b2c, sc2, sh2, w3, b3c), 128,
                         [_full_spec((64, CIN)), _full_spec((64, 1)),
                          _full_spec((64, 1)), _full_spec((64, 1)),
                          _full_spec((64, 64)), _full_spec((64, 1)),
                          _full_spec((64, 1)), _full_spec((64, 1)),
                          _full_spec((128, 64)), _full_spec((128, 1))])
    sc3, sh3 = _finalize(s3, q3, g3, bt3)

    x3d = x_flat.reshape(CIN, K, B * S)
    pooled = pl.pallas_call(
        _p4_body,
        grid=(B * S // _LB,),
        in_specs=[pl.BlockSpec((CIN, K, _LB), lambda i: (0, 0, i)),
                  _full_spec((64, CIN)), _full_spec((64, 1)),
                  _full_spec((64, 1)), _full_spec((64, 1)),
                  _full_spec((64, 64)), _full_spec((64, 1)),
                  _full_spec((64, 1)), _full_spec((64, 1)),
                  _full_spec((128, 64)), _full_spec((128, 1)),
                  _full_spec((128, 1)), _full_spec((128, 1))],
        out_specs=_col_spec(128, _LB),
        out_shape=jax.ShapeDtypeStruct((128, B * S), jnp.float32),
    )(x3d, w1, b1c, sc1, sh1, w2, b2c, sc2, sh2, w3, b3c, sc3, sh3)
    return pooled


def kernel(xyz, points, w1, b1, g1, bt1, w2, b2, g2, bt2, w3, b3, g3, bt3):
    new_xyz = _fps_call(xyz)
    grouped = _group_call()(xyz.reshape(-1), points.reshape(-1),
                            new_xyz.reshape(-1))
    x_flat = grouped.reshape(CIN, M)
    pooled = _mlp_call(x_flat, w1, b1, g1, bt1, w2, b2, g2, bt2,
                       w3, b3, g3, bt3)
    new_points_out = pooled.reshape(128, B, S).transpose(1, 0, 2)
    return new_xyz, new_points_out
